# baseline (speedup 1.0000x reference)
import functools

import jax
import jax.numpy as jnp
from jax import lax
from jax.experimental import pallas as pl
from jax.experimental.pallas import tpu as pltpu
from jax.experimental.pallas import tpu_sc as plsc

VOCAB = 1000000
EMB = 32
HID = 128
OUT = 2
B = 16384
S = 200
SP = 256

NC = 2
NS = 16
NW = NC * NS
BPW = B // NW
G = 8
ROWS = G * S
NG = BPW // G
RD = 4

TRC = 32768
TRQ = TRC // 8
NBLK = (VOCAB + TRC - 1) // TRC
NREC = NBLK * TRC


def _tr_block(x_ref, o_ref):
    x = x_ref[...]
    packs = []
    for q in range(8):
        s = x[:, q * TRQ:(q + 1) * TRQ].astype(jnp.bfloat16)
        lo = jax.lax.bitcast_convert_type(s[0:16], jnp.uint16).astype(jnp.uint32)
        hi = jax.lax.bitcast_convert_type(s[16:32], jnp.uint16).astype(jnp.uint32)
        packs.append((lo | (hi << 16)).astype(jnp.int32))
    z = jnp.concatenate(packs, axis=0)
    o_ref[...] = jnp.transpose(z)


def _pack_table_tc(embT):
    return pl.pallas_call(
        _tr_block,
        grid=(NBLK,),
        in_specs=[pl.BlockSpec((EMB, TRC), lambda i: (0, i))],
        out_specs=pl.BlockSpec((TRQ, 128), lambda i: (i, 0)),
        out_shape=jax.ShapeDtypeStruct((NBLK * TRQ, 128), jnp.int32),
    )(embT)


def _pooled_sums_sc(rec_flat, table):
    mesh = plsc.VectorSubcoreMesh(
        core_axis_name="c", subcore_axis_name="s", num_cores=NC, num_subcores=NS
    )

    @functools.partial(
        pl.kernel,
        out_type=jax.ShapeDtypeStruct((B, EMB), jnp.float32),
        mesh=mesh,
        compiler_params=pltpu.CompilerParams(
            use_tc_tiling_on_sc=False, needs_layout_passes=False
        ),
        scratch_types=(
            [pltpu.VMEM((ROWS,), jnp.int32) for _ in range(RD)]
            + [pltpu.VMEM((ROWS, 16), jnp.int32) for _ in range(RD)]
            + [pltpu.VMEM((BPW, EMB), jnp.float32)]
            + [pltpu.SemaphoreType.DMA for _ in range(2 * RD)]
        ),
    )
    def k(rec_hbm, tab_hbm, out_hbm, *scr):
        idx_refs = scr[0:RD]
        row_refs = scr[RD:2 * RD]
        outb = scr[2 * RD]
        isems = scr[2 * RD + 1:3 * RD + 1]
        rsems = scr[3 * RD + 1:4 * RD + 1]
        wid = lax.axis_index("c") * NS + lax.axis_index("s")
        row_base = wid * BPW

        def idx_start(g, buf):
            for i in range(G):
                b = row_base + g * G + i
                rb = b // 8
                rr = b % 8
                pltpu.async_copy(rec_hbm.at[rb, 0, rr, :],
                                 idx_refs[buf].at[pl.ds(i * S, 128)],
                                 isems[buf])
                pltpu.async_copy(rec_hbm.at[rb, 1, rr, pl.ds(0, S - 128)],
                                 idx_refs[buf].at[pl.ds(i * S + 128, S - 128)],
                                 isems[buf])

        def idx_wait(buf):
            for i in range(G):
                pltpu.make_async_copy(
                    rec_hbm.at[0, 0, 0, :],
                    idx_refs[buf].at[pl.ds(0, 128)], isems[buf]
                ).wait()
                pltpu.make_async_copy(
                    rec_hbm.at[0, 1, 0, pl.ds(0, S - 128)],
                    idx_refs[buf].at[pl.ds(0, S - 128)], isems[buf]
                ).wait()

        def gather_start(buf):
            pltpu.async_copy(tab_hbm.at[idx_refs[buf]], row_refs[buf], rsems[buf])

        def gather_wait(buf):
            pltpu.make_async_copy(
                tab_hbm.at[idx_refs[buf]], row_refs[buf], rsems[buf]
            ).wait()

        def reduce(g, buf):
            rows = row_refs[buf]
            for i in range(G):
                def body(j, accs):
                    a0, a1 = accs
                    base = i * S + j * 8
                    v = [
                        plsc.bitcast(rows[base + u, 0:16], jnp.bfloat16)
                        for u in range(8)
                    ]
                    w = [v[2 * u] + v[2 * u + 1] for u in range(4)]
                    x2 = [w[0] + w[1], w[2] + w[3]]
                    t = x2[0] + x2[1]
                    lo, hi = plsc.unpack(t, format=plsc.PackFormat.INTERLEAVED)
                    return (a0 + lo, a1 + hi)
                a0, a1 = lax.fori_loop(
                    0, S // 8, body,
                    (jnp.zeros((16,), jnp.float32), jnp.zeros((16,), jnp.float32)),
                )
                outb[g * G + i, 0:16] = a0
                outb[g * G + i, 16:32] = a1

        for b in range(RD):
            idx_start(b, b)
        for b in range(RD - 1):
            idx_wait(b)
            gather_start(b)

        def phase(g, b, issue_idx, issue_gather):
            gather_wait(b)
            if issue_idx:
                idx_start(g + RD, b)
            if issue_gather:
                idx_wait((b + RD - 1) % RD)
                gather_start((b + RD - 1) % RD)
            reduce(g, b)

        def step(p, carry):
            for kk in range(RD):
                phase(RD * p + kk, kk, True, True)
            return carry

        lax.fori_loop(0, NG // RD - 1, step, 0)

        g0 = NG - RD
        phase(g0, 0, False, True)
        for kk in range(1, RD):
            phase(g0 + kk, kk, False, False)

        pltpu.sync_copy(outb, out_hbm.at[pl.ds(wid * BPW, BPW)])

    return k(rec_flat, table)


def _mlp_block(x_ref, w1t_ref, b1_ref, w2t_ref, b2_ref, o_ref):
    x = x_ref[...] * (1.0 / S)
    h = jnp.dot(x, w1t_ref[...], preferred_element_type=jnp.float32) + b1_ref[...]
    h = jnp.maximum(h, 0.0)
    o_ref[...] = (
        jnp.dot(h, w2t_ref[...], preferred_element_type=jnp.float32) + b2_ref[...]
    )


def _mlp_tc(pooled_sums, W1, b1, W2, b2):
    blk = 2048
    grid = (B // blk,)
    return pl.pallas_call(
        _mlp_block,
        grid=grid,
        in_specs=[
            pl.BlockSpec((blk, EMB), lambda i: (i, 0)),
            pl.BlockSpec((EMB, HID), lambda i: (0, 0)),
            pl.BlockSpec((1, HID), lambda i: (0, 0)),
            pl.BlockSpec((HID, OUT), lambda i: (0, 0)),
            pl.BlockSpec((1, OUT), lambda i: (0, 0)),
        ],
        out_specs=pl.BlockSpec((blk, OUT), lambda i: (i, 0)),
        out_shape=jax.ShapeDtypeStruct((B, OUT), jnp.float32),
    )(pooled_sums, W1.T, b1.reshape(1, HID), W2.T, b2.reshape(1, OUT))


def kernel(text, lengths, emb, W1, b1, W2, b2):
    del lengths
    tf = jnp.pad(text.astype(jnp.int32), ((0, 0), (0, SP - S)))
    rec = (tf & jnp.int32(-TRC)) | ((tf & jnp.int32(TRQ - 1)) << 3) | (
        (tf >> (TRQ.bit_length() - 1)) & jnp.int32(7)
    )
    rec = rec.reshape(B // 8, 8, SP // 128, 128).transpose(0, 2, 1, 3)
    packed = _pack_table_tc(emb.T)
    table = packed.reshape(-1).reshape(NREC, 16)
    pooled_sums = _pooled_sums_sc(rec, table)
    return _mlp_tc(pooled_sums, W1, b1, W2, b2)


# speedup vs baseline: 1.0262x; 1.0262x over previous
import functools

import jax
import jax.numpy as jnp
from jax import lax
from jax.experimental import pallas as pl
from jax.experimental.pallas import tpu as pltpu
from jax.experimental.pallas import tpu_sc as plsc

VOCAB = 1000000
EMB = 32
HID = 128
OUT = 2
B = 16384
S = 200
SP = 256

NC = 2
NS = 16
NW = NC * NS
BPW = B // NW
G = 4
ROWS = G * S
NG = BPW // G
RD = 4

TRC = 32768
TRQ = TRC // 8
NBLK = (VOCAB + TRC - 1) // TRC
NREC = NBLK * TRC


def _tr_block(x_ref, o_ref):
    x = x_ref[...]
    packs = []
    for q in range(8):
        s = x[:, q * TRQ:(q + 1) * TRQ].astype(jnp.bfloat16)
        lo = jax.lax.bitcast_convert_type(s[0:16], jnp.uint16).astype(jnp.uint32)
        hi = jax.lax.bitcast_convert_type(s[16:32], jnp.uint16).astype(jnp.uint32)
        packs.append((lo | (hi << 16)).astype(jnp.int32))
    z = jnp.concatenate(packs, axis=0)
    o_ref[...] = jnp.transpose(z)


def _pack_table_tc(embT):
    return pl.pallas_call(
        _tr_block,
        grid=(NBLK,),
        in_specs=[pl.BlockSpec((EMB, TRC), lambda i: (0, i))],
        out_specs=pl.BlockSpec((TRQ, 128), lambda i: (i, 0)),
        out_shape=jax.ShapeDtypeStruct((NBLK * TRQ, 128), jnp.int32),
    )(embT)


def _pooled_sums_sc(rec_flat, table):
    mesh = plsc.VectorSubcoreMesh(
        core_axis_name="c", subcore_axis_name="s", num_cores=NC, num_subcores=NS
    )

    @functools.partial(
        pl.kernel,
        out_type=jax.ShapeDtypeStruct((B, EMB), jnp.float32),
        mesh=mesh,
        compiler_params=pltpu.CompilerParams(
            use_tc_tiling_on_sc=False, needs_layout_passes=False
        ),
        scratch_types=(
            [pltpu.VMEM((ROWS,), jnp.int32) for _ in range(RD)]
            + [pltpu.VMEM((ROWS, 16), jnp.int32) for _ in range(RD)]
            + [pltpu.VMEM((BPW, EMB), jnp.float32)]
            + [pltpu.SemaphoreType.DMA for _ in range(2 * RD)]
        ),
    )
    def k(rec_hbm, tab_hbm, out_hbm, *scr):
        idx_refs = scr[0:RD]
        row_refs = scr[RD:2 * RD]
        outb = scr[2 * RD]
        isems = scr[2 * RD + 1:3 * RD + 1]
        rsems = scr[3 * RD + 1:4 * RD + 1]
        wid = lax.axis_index("c") * NS + lax.axis_index("s")
        row_base = wid * BPW

        def idx_start(g, buf):
            for i in range(G):
                b = row_base + g * G + i
                rb = b // 8
                rr = b % 8
                pltpu.async_copy(rec_hbm.at[rb, 0, rr, :],
                                 idx_refs[buf].at[pl.ds(i * S, 128)],
                                 isems[buf])
                pltpu.async_copy(rec_hbm.at[rb, 1, rr, pl.ds(0, S - 128)],
                                 idx_refs[buf].at[pl.ds(i * S + 128, S - 128)],
                                 isems[buf])

        def idx_wait(buf):
            for i in range(G):
                pltpu.make_async_copy(
                    rec_hbm.at[0, 0, 0, :],
                    idx_refs[buf].at[pl.ds(0, 128)], isems[buf]
                ).wait()
                pltpu.make_async_copy(
                    rec_hbm.at[0, 1, 0, pl.ds(0, S - 128)],
                    idx_refs[buf].at[pl.ds(0, S - 128)], isems[buf]
                ).wait()

        def gather_start(buf):
            pltpu.async_copy(tab_hbm.at[idx_refs[buf]], row_refs[buf], rsems[buf])

        def gather_wait(buf):
            pltpu.make_async_copy(
                tab_hbm.at[idx_refs[buf]], row_refs[buf], rsems[buf]
            ).wait()

        def reduce(g, buf):
            rows = row_refs[buf]
            for i in range(G):
                def body(j, accs):
                    a0, a1 = accs
                    base = i * S + j * 8
                    v = [
                        plsc.bitcast(rows[base + u, 0:16], jnp.bfloat16)
                        for u in range(8)
                    ]
                    w = [v[2 * u] + v[2 * u + 1] for u in range(4)]
                    x2 = [w[0] + w[1], w[2] + w[3]]
                    t = x2[0] + x2[1]
                    lo, hi = plsc.unpack(t, format=plsc.PackFormat.INTERLEAVED)
                    return (a0 + lo, a1 + hi)
                a0, a1 = lax.fori_loop(
                    0, S // 8, body,
                    (jnp.zeros((16,), jnp.float32), jnp.zeros((16,), jnp.float32)),
                )
                outb[g * G + i, 0:16] = a0
                outb[g * G + i, 16:32] = a1

        for b in range(RD):
            idx_start(b, b)
        for b in range(RD - 1):
            idx_wait(b)
            gather_start(b)

        def phase(g, b, issue_idx, issue_gather):
            gather_wait(b)
            if issue_idx:
                idx_start(g + RD, b)
            if issue_gather:
                idx_wait((b + RD - 1) % RD)
                gather_start((b + RD - 1) % RD)
            reduce(g, b)

        def step(p, carry):
            for kk in range(RD):
                phase(RD * p + kk, kk, True, True)
            return carry

        lax.fori_loop(0, NG // RD - 1, step, 0)

        g0 = NG - RD
        phase(g0, 0, False, True)
        for kk in range(1, RD):
            phase(g0 + kk, kk, False, False)

        pltpu.sync_copy(outb, out_hbm.at[pl.ds(wid * BPW, BPW)])

    return k(rec_flat, table)


def _mlp_block(x_ref, w1t_ref, b1_ref, w2t_ref, b2_ref, o_ref):
    x = x_ref[...] * (1.0 / S)
    h = jnp.dot(x, w1t_ref[...], preferred_element_type=jnp.float32) + b1_ref[...]
    h = jnp.maximum(h, 0.0)
    o_ref[...] = (
        jnp.dot(h, w2t_ref[...], preferred_element_type=jnp.float32) + b2_ref[...]
    )


def _mlp_tc(pooled_sums, W1, b1, W2, b2):
    x4 = pooled_sums.reshape(B // 4, 4 * EMB)
    eye4 = jnp.eye(4, dtype=jnp.float32)
    w1bd = jnp.kron(eye4, W1.T)
    b1t = jnp.tile(b1, 4).reshape(1, 4 * HID)
    w2bd = jnp.kron(eye4, W2.T)
    b2t = jnp.tile(b2, 4).reshape(1, 4 * OUT)
    blk = 1024
    grid = (B // 4 // blk,)
    o4 = pl.pallas_call(
        _mlp_block,
        grid=grid,
        in_specs=[
            pl.BlockSpec((blk, 4 * EMB), lambda i: (i, 0)),
            pl.BlockSpec((4 * EMB, 4 * HID), lambda i: (0, 0)),
            pl.BlockSpec((1, 4 * HID), lambda i: (0, 0)),
            pl.BlockSpec((4 * HID, 4 * OUT), lambda i: (0, 0)),
            pl.BlockSpec((1, 4 * OUT), lambda i: (0, 0)),
        ],
        out_specs=pl.BlockSpec((blk, 4 * OUT), lambda i: (i, 0)),
        out_shape=jax.ShapeDtypeStruct((B // 4, 4 * OUT), jnp.float32),
    )(x4, w1bd, b1t, w2bd, b2t)
    return o4.reshape(B, OUT)


def kernel(text, lengths, emb, W1, b1, W2, b2):
    del lengths
    tf = jnp.pad(text.astype(jnp.int32), ((0, 0), (0, SP - S)))
    rec = (tf & jnp.int32(-TRC)) | ((tf & jnp.int32(TRQ - 1)) << 3) | (
        (tf >> (TRQ.bit_length() - 1)) & jnp.int32(7)
    )
    rec = rec.reshape(B // 8, 8, SP // 128, 128).transpose(0, 2, 1, 3)
    packed = _pack_table_tc(emb.T)
    table = packed.reshape(-1).reshape(NREC, 16)
    pooled_sums = _pooled_sums_sc(rec, table)
    return _mlp_tc(pooled_sums, W1, b1, W2, b2)
